# Initial kernel scaffold; baseline (speedup 1.0000x reference)
import jax
import jax.numpy as jnp
from jax.experimental import pallas as pl


def kernel(x, W_gate):
    raise NotImplementedError("write your pallas kernel here")


# speedup vs baseline: 1.4136x; 1.4136x over previous
import jax
import jax.numpy as jnp
from jax.experimental import pallas as pl
from jax.experimental.pallas import tpu as pltpu

_DIM = 4096
_E = 64
_TOPK = 8
_T = 16384
_BT = 1024
_NB = _T // _BT


def _router_kernel(x_ref, w_ref,
                   idx_ref, wts_ref, bal_ref, z_ref, usage_ref,
                   psum_ref, fcnt_ref, lse2_ref):
    i = pl.program_id(0)

    @pl.when(i == 0)
    def _init():
        psum_ref[...] = jnp.zeros_like(psum_ref)
        fcnt_ref[...] = jnp.zeros_like(fcnt_ref)
        lse2_ref[...] = jnp.zeros_like(lse2_ref)

    x = x_ref[...]
    w = w_ref[...]
    logits = jax.lax.dot_general(
        x, w, (((1,), (1,)), ((), ())),
        preferred_element_type=jnp.float32)

    m = jnp.max(logits, axis=1, keepdims=True)
    e = jnp.exp(logits - m)
    s = jnp.sum(e, axis=1, keepdims=True)
    probs = e / s
    lse = m + jnp.log(s)
    lse2_ref[...] += jnp.sum(lse * lse).reshape(1, 1)

    iota_e = jax.lax.broadcasted_iota(jnp.int32, (_BT, _E), 1)
    p = probs
    vals = []
    idxs = []
    top1 = None
    for k in range(_TOPK):
        v = jnp.max(p, axis=1, keepdims=True)
        idx = jnp.min(jnp.where(p == v, iota_e, _E), axis=1, keepdims=True)
        vals.append(v)
        idxs.append(idx)
        if k == 0:
            top1 = idx
        p = jnp.where(iota_e == idx, -1.0, p)

    tv = jnp.concatenate(vals, axis=1)
    ti = jnp.concatenate(idxs, axis=1)
    wts = tv / (jnp.sum(tv, axis=1, keepdims=True) + 1e-8)
    idx_ref[...] = ti
    wts_ref[...] = wts

    psum_ref[...] += jnp.sum(probs, axis=0, keepdims=True)
    fcnt_ref[...] += jnp.sum((top1 == iota_e).astype(jnp.float32),
                             axis=0, keepdims=True)

    @pl.when(i == _NB - 1)
    def _fini():
        f = fcnt_ref[...] / _T
        pmean = psum_ref[...] / _T
        bal_ref[...] = (_E * jnp.sum(f * pmean)).reshape(1, 1)
        z_ref[...] = lse2_ref[...] / _T
        usage_ref[...] = f


def kernel(x, W_gate):
    idx, wts, bal, z, usage = pl.pallas_call(
        _router_kernel,
        grid=(_NB,),
        in_specs=[
            pl.BlockSpec((_BT, _DIM), lambda i: (i, 0)),
            pl.BlockSpec((_E, _DIM), lambda i: (0, 0)),
        ],
        out_specs=[
            pl.BlockSpec((_BT, _TOPK), lambda i: (i, 0)),
            pl.BlockSpec((_BT, _TOPK), lambda i: (i, 0)),
            pl.BlockSpec((1, 1), lambda i: (0, 0)),
            pl.BlockSpec((1, 1), lambda i: (0, 0)),
            pl.BlockSpec((1, _E), lambda i: (0, 0)),
        ],
        out_shape=[
            jax.ShapeDtypeStruct((_T, _TOPK), jnp.int32),
            jax.ShapeDtypeStruct((_T, _TOPK), jnp.float32),
            jax.ShapeDtypeStruct((1, 1), jnp.float32),
            jax.ShapeDtypeStruct((1, 1), jnp.float32),
            jax.ShapeDtypeStruct((1, _E), jnp.float32),
        ],
        scratch_shapes=[
            pltpu.VMEM((1, _E), jnp.float32),
            pltpu.VMEM((1, _E), jnp.float32),
            pltpu.VMEM((1, 1), jnp.float32),
        ],
    )(x, W_gate)
    return (idx, wts, bal[0, 0], z[0, 0], usage[0])


# speedup vs baseline: 1.4890x; 1.0533x over previous
import jax
import jax.numpy as jnp
from jax.experimental import pallas as pl
from jax.experimental.pallas import tpu as pltpu

_DIM = 4096
_E = 64
_TOPK = 8
_T = 16384
_BT = 1024
_NB = _T // _BT


def _router_kernel(x_ref, w_ref,
                   idx_ref, wts_ref, bal_ref, z_ref, usage_ref,
                   psum_ref, fcnt_ref, lse2_ref):
    i = pl.program_id(0)

    @pl.when(i == 0)
    def _init():
        psum_ref[...] = jnp.zeros_like(psum_ref)
        fcnt_ref[...] = jnp.zeros_like(fcnt_ref)
        lse2_ref[...] = jnp.zeros_like(lse2_ref)

    x = x_ref[...]
    w = w_ref[...]
    logits = jax.lax.dot_general(
        x, w, (((1,), (1,)), ((), ())),
        preferred_element_type=jnp.float32)

    m = jnp.max(logits, axis=1, keepdims=True)
    e = jnp.exp(logits - m)
    s = jnp.sum(e, axis=1, keepdims=True)
    probs = e / s
    lse = m + jnp.log(s)
    lse2_ref[...] += jnp.sum(lse * lse).reshape(1, 1)

    iota_e = jax.lax.broadcasted_iota(jnp.int32, (_BT, _E), 1)
    bits = jax.lax.bitcast_convert_type(probs, jnp.int32)
    keys = jax.lax.bitwise_or(jax.lax.bitwise_and(bits, ~63),
                              (_E - 1) - iota_e)
    vals = []
    idxs = []
    top1 = None
    for k in range(_TOPK):
        kmax = jnp.max(keys, axis=1, keepdims=True)
        idx = (_E - 1) - jax.lax.bitwise_and(kmax, _E - 1)
        v = jax.lax.bitcast_convert_type(
            jax.lax.bitwise_and(kmax, ~63), jnp.float32)
        vals.append(v)
        idxs.append(idx)
        if k == 0:
            top1 = idx
        keys = jnp.where(keys == kmax, jnp.int32(-1), keys)

    tv = jnp.concatenate(vals, axis=1)
    ti = jnp.concatenate(idxs, axis=1)
    wts = tv / (jnp.sum(tv, axis=1, keepdims=True) + 1e-8)
    idx_ref[...] = ti
    wts_ref[...] = wts

    psum_ref[...] += jnp.sum(probs, axis=0, keepdims=True)
    fcnt_ref[...] += jnp.sum((top1 == iota_e).astype(jnp.float32),
                             axis=0, keepdims=True)

    @pl.when(i == _NB - 1)
    def _fini():
        f = fcnt_ref[...] / _T
        pmean = psum_ref[...] / _T
        bal_ref[...] = (_E * jnp.sum(f * pmean)).reshape(1, 1)
        z_ref[...] = lse2_ref[...] / _T
        usage_ref[...] = f


def kernel(x, W_gate):
    idx, wts, bal, z, usage = pl.pallas_call(
        _router_kernel,
        grid=(_NB,),
        in_specs=[
            pl.BlockSpec((_BT, _DIM), lambda i: (i, 0)),
            pl.BlockSpec((_E, _DIM), lambda i: (0, 0)),
        ],
        out_specs=[
            pl.BlockSpec((_BT, _TOPK), lambda i: (i, 0)),
            pl.BlockSpec((_BT, _TOPK), lambda i: (i, 0)),
            pl.BlockSpec((1, 1), lambda i: (0, 0)),
            pl.BlockSpec((1, 1), lambda i: (0, 0)),
            pl.BlockSpec((1, _E), lambda i: (0, 0)),
        ],
        out_shape=[
            jax.ShapeDtypeStruct((_T, _TOPK), jnp.int32),
            jax.ShapeDtypeStruct((_T, _TOPK), jnp.float32),
            jax.ShapeDtypeStruct((1, 1), jnp.float32),
            jax.ShapeDtypeStruct((1, 1), jnp.float32),
            jax.ShapeDtypeStruct((1, _E), jnp.float32),
        ],
        scratch_shapes=[
            pltpu.VMEM((1, _E), jnp.float32),
            pltpu.VMEM((1, _E), jnp.float32),
            pltpu.VMEM((1, 1), jnp.float32),
        ],
    )(x, W_gate)
    return (idx, wts, bal[0, 0], z[0, 0], usage[0])
